# Initial kernel scaffold; baseline (speedup 1.0000x reference)
import jax
import jax.numpy as jnp
from jax.experimental import pallas as pl


def kernel(mem, edge_feat, W_g, a, W_e, W_uc, W_un, W_p, W_1, W_2, src_idxs, dst_idxs, neg_idxs, edge_idxs, timestamps, nbrs_src, nbrs_dst, ts_src, ts_dst, up_nbrs_src, up_nbrs_dst, up_ts_src, up_ts_dst):
    raise NotImplementedError("write your pallas kernel here")


# speedup vs baseline: 1.8521x; 1.8521x over previous
import functools

import jax
import jax.numpy as jnp
from jax import lax
from jax.experimental import pallas as pl
from jax.experimental.pallas import tpu as pltpu
from jax.experimental.pallas import tpu_sc as plsc

N_NODES = 100000
EMB = 128
MSG = 128
K1 = 16
K2 = 16
B = 4096
TOT_TIME = 100.0
GAMMA = 1.0
EPS = 1e-10

_INFO = plsc.get_sparse_core_info()
NC = _INFO.num_cores
NS = _INFO.num_subcores
NW = NC * NS
NOWN = (N_NODES + NW - 1) // NW
NPAD = N_NODES + NW
PR_LEN = ((NOWN + 15) // 16) * 16
PR_ITERS = PR_LEN // 16
RC = 512
WL = ((NOWN + RC - 1) // RC) * RC
NCH_MAX = WL // RC
IC = 4096

_mesh = functools.partial(
    plsc.VectorSubcoreMesh, core_axis_name="c", subcore_axis_name="s")
_SC_PARAMS = pltpu.CompilerParams(needs_layout_passes=False)
_PREC = jax.lax.Precision.DEFAULT


def _wid():
    return lax.axis_index("s") * NC + lax.axis_index("c")


def _pick_chunk(per_w, row_bytes):
    target = max(8, (128 * 1024) // row_bytes)
    ch = min(per_w, target)
    while ch > 8 and (per_w % ch or ch % 8):
        ch -= 1
    return ch


def _sc_gather(table, idx, n_rows, dim):
    per_w = n_rows // NW
    ch = _pick_chunk(per_w, dim * 4)
    chunks = per_w // ch

    @functools.partial(
        pl.kernel,
        out_type=jax.ShapeDtypeStruct((n_rows, dim), jnp.float32),
        mesh=_mesh(),
        compiler_params=_SC_PARAMS,
        scratch_types=[
            pltpu.VMEM((ch,), jnp.int32),
            pltpu.VMEM((ch, dim), jnp.float32),
            pltpu.SemaphoreType.DMA,
        ],
        name=f"sc_gather_{n_rows}x{dim}",
    )
    def k(table_hbm, idx_hbm, out_hbm, idxv, rowsv, sem):
        base = _wid() * per_w

        def body(j, carry):
            off = base + j * ch
            pltpu.sync_copy(idx_hbm.at[pl.ds(off, ch)], idxv)
            pltpu.async_copy(table_hbm.at[idxv], rowsv, sem).wait()
            pltpu.sync_copy(rowsv, out_hbm.at[pl.ds(off, ch)])
            return carry

        lax.fori_loop(0, chunks, body, 0)

    return k(table, idx)


def _sc_scatter(mem_ref, idx, val, n_upd):
    ic = min(IC, n_upd)
    chunks = n_upd // ic

    @functools.partial(
        pl.kernel,
        out_type=(),
        mesh=_mesh(),
        compiler_params=_SC_PARAMS,
        scratch_types=[
            pltpu.VMEM((ic,), jnp.int32),
            pltpu.VMEM((PR_LEN,), jnp.int32),
            pltpu.VMEM((WL,), jnp.int32),
            pltpu.VMEM((WL,), jnp.int32),
            pltpu.VMEM((RC,), jnp.int32),
            pltpu.VMEM((RC,), jnp.int32),
            pltpu.VMEM((RC, EMB), jnp.float32),
            pltpu.SemaphoreType.DMA,
        ],
        name=f"sc_scatter_{n_upd}",
    )
    def k(mem_hbm, idx_hbm, val_hbm, idxv, priov, wposv, wnodev,
          pos_dma, node_dma, rowsv, sem):
        wid = _wid()
        base = wid * NOWN
        neg1 = jnp.full((16,), -1, jnp.int32)

        def initb(i, c):
            priov[pl.ds(i * 16, 16)] = neg1
            return c

        lax.fori_loop(0, PR_ITERS, initb, 0)

        def c_body(c, carry):
            pltpu.sync_copy(idx_hbm.at[pl.ds(c * ic, ic)], idxv)

            def s_body(s, carry2):
                v = idxv[pl.ds(s * 16, 16)]
                pos = c * ic + s * 16 + lax.iota(jnp.int32, 16)
                rel = v - base
                own = (rel >= 0) & (rel < NOWN)
                rel_s = jnp.where(own, rel, 0)
                plsc.store_scatter(priov, [rel_s], pos, mask=own)
                return carry2

            lax.fori_loop(0, ic // 16, s_body, 0)
            return carry

        lax.fori_loop(0, chunks, c_body, 0)

        pad_pos = jnp.full((16,), (wid * 997) % n_upd, jnp.int32)
        dump = jnp.full((16,), N_NODES + wid, jnp.int32)

        def pf(i, c):
            wposv[pl.ds(i * 16, 16)] = pad_pos
            return c

        lax.fori_loop(0, WL // 16, pf, 0)

        def pf2(i, c):
            wnodev[pl.ds(i * 16, 16)] = dump
            return c

        lax.fori_loop(0, WL // 16, pf2, 0)

        def p2(s, cnt):
            p = priov[pl.ds(s * 16, 16)]
            m = p >= 0
            mi = m.astype(jnp.int32)
            pref = plsc.cumsum(mi)
            dest = jnp.where(m, cnt + pref - 1, 0)
            plsc.store_scatter(wposv, [dest], p, mask=m)
            node = s * 16 + lax.iota(jnp.int32, 16) + base
            plsc.store_scatter(wnodev, [dest], node, mask=m)
            return cnt + jnp.sum(mi)

        cnt = lax.fori_loop(0, PR_ITERS, p2, 0)

        nch = (cnt + RC - 1) // RC

        def p3(j, c):
            def stage(i, c2):
                pos_dma[pl.ds(i * 16, 16)] = wposv[pl.ds(j * RC + i * 16, 16)]
                node_dma[pl.ds(i * 16, 16)] = wnodev[pl.ds(j * RC + i * 16, 16)]
                return c2

            lax.fori_loop(0, RC // 16, stage, 0)
            pltpu.async_copy(val_hbm.at[pos_dma], rowsv, sem).wait()
            pltpu.async_copy(rowsv, mem_hbm.at[node_dma], sem).wait()
            return c

        lax.fori_loop(0, nch, p3, 0)

    k(mem_ref, idx, val)


def _damp(timestamps, ts):
    interval = (timestamps[:, None] - ts) / TOT_TIME
    return 1.0 / (1.0 + GAMMA * interval)


RB = 128


def _d1_kernel(nes_ref, ned_ref, ms_ref, md_ref, er_ref, ds_ref, dd_ref,
               wg_ref, a2_ref, we_ref, wuc_ref, wp_ref,
               upv_ref, hp_ref):
    wg = wg_ref[...]

    def phase(ne_ref, m_ref, d_ref):
        ne3 = ne_ref[...].reshape(RB, K1, EMB) * d_ref[...][..., None]
        hn = jnp.dot(ne3.reshape(RB * K1, EMB), wg,
                     preferred_element_type=jnp.float32, precision=_PREC)
        hn3 = hn.reshape(RB, K1, MSG // 2)
        hc = jnp.dot(m_ref[...], wg, preferred_element_type=jnp.float32, precision=_PREC)
        hcb = jnp.broadcast_to(hc[:, None, :], (RB, K1, MSG // 2))
        h_in = jnp.concatenate([hcb, hn3], axis=2).reshape(RB * K1, MSG)
        l3 = jnp.dot(h_in, a2_ref[...],
                     preferred_element_type=jnp.float32, precision=_PREC).reshape(RB, K1, 1)
        l3 = jnp.where(l3 >= 0, l3, 0.2 * l3)
        mx = jnp.max(l3, axis=1, keepdims=True)
        e = jnp.exp(l3 - mx)
        att3 = e / jnp.sum(e, axis=1, keepdims=True)
        return jnp.sum(hn3 * att3, axis=1)

    msg_s = phase(nes_ref, ms_ref, ds_ref)
    msg_d = phase(ned_ref, md_ref, dd_ref)
    hnode = jnp.tanh(jnp.concatenate([msg_s, msg_d], axis=1))
    he = jnp.tanh(jnp.dot(er_ref[...], we_ref[...],
                          preferred_element_type=jnp.float32, precision=_PREC))
    h = jnp.concatenate([hnode, he], axis=1)
    wuc = wuc_ref[...]
    upv_ref[0] = jnp.tanh(jnp.dot(
        jnp.concatenate([ms_ref[...], h], axis=1), wuc,
        preferred_element_type=jnp.float32, precision=_PREC))
    upv_ref[1] = jnp.tanh(jnp.dot(
        jnp.concatenate([md_ref[...], h], axis=1), wuc,
        preferred_element_type=jnp.float32, precision=_PREC))
    hp_ref[...] = jnp.dot(h, wp_ref[...], preferred_element_type=jnp.float32, precision=_PREC)


def _d1(rows1, erows, damp_s, damp_d, W_g, a, W_e, W_uc, W_p):
    grid = (B // RB,)
    nb = RB * K1
    specs = [
        pl.BlockSpec((nb, EMB), lambda i: (i, 0)),
        pl.BlockSpec((nb, EMB), lambda i: (B * K1 // nb + i, 0)),
        pl.BlockSpec((RB, EMB), lambda i: (2 * B * K1 // RB + i, 0)),
        pl.BlockSpec((RB, EMB), lambda i: (2 * B * K1 // RB + B // RB + i, 0)),
        pl.BlockSpec((RB, 16), lambda i: (i, 0)),
        pl.BlockSpec((RB, K1), lambda i: (i, 0)),
        pl.BlockSpec((RB, K1), lambda i: (i, 0)),
        pl.BlockSpec((EMB, MSG // 2), lambda i: (0, 0)),
        pl.BlockSpec((MSG, 1), lambda i: (0, 0)),
        pl.BlockSpec((16, MSG), lambda i: (0, 0)),
        pl.BlockSpec((EMB + 2 * MSG, EMB), lambda i: (0, 0)),
        pl.BlockSpec((2 * MSG, MSG), lambda i: (0, 0)),
    ]
    out_specs = [
        pl.BlockSpec((2, RB, EMB), lambda i: (0, i, 0)),
        pl.BlockSpec((RB, MSG), lambda i: (i, 0)),
    ]
    out_shape = [
        jax.ShapeDtypeStruct((2, B, EMB), jnp.float32),
        jax.ShapeDtypeStruct((B, MSG), jnp.float32),
    ]
    a2 = a.reshape(-1, 1)
    return pl.pallas_call(
        _d1_kernel, grid=grid, in_specs=specs, out_specs=out_specs,
        out_shape=out_shape, name="d1",
    )(rows1, rows1, rows1, rows1, erows, damp_s, damp_d,
      W_g, a2, W_e, W_uc, W_p)


def _d2_kernel(g_ref, hp_ref, d_ref, w1_ref, w2_ref, wun_ref, val_ref):
    g = g_ref[...]
    g3 = g.reshape(RB, K2, EMB)
    ne3 = g3 * d_ref[...][..., None]
    h1 = jnp.broadcast_to(hp_ref[...][:, None, :], (RB, K2, MSG))
    h2 = h1 * ne3
    nrm = jnp.sqrt(jnp.sum(h2 * h2, axis=2, keepdims=True))
    h2n = h2 / (nrm + EPS)
    sc = jnp.sum(h2n, axis=2)
    mx = jnp.max(sc, axis=1, keepdims=True)
    e = jnp.exp(sc - mx)
    att = e / jnp.sum(e, axis=1, keepdims=True)
    changed = h1 * att[..., None]
    ch2d = changed.reshape(RB * K2, MSG)
    x = jnp.concatenate([ne3.reshape(RB * K2, EMB), ch2d], axis=1)
    x = jnp.dot(x, w1_ref[...], preferred_element_type=jnp.float32, precision=_PREC)
    x = jnp.maximum(x, 0.0)
    z = jnp.dot(x, w2_ref[...], preferred_element_type=jnp.float32, precision=_PREC)
    changed2 = jnp.tanh(jnp.dot(
        jnp.concatenate([g, ch2d], axis=1), wun_ref[...],
        preferred_element_type=jnp.float32, precision=_PREC))
    mask = (z >= 0.0).astype(jnp.float32)
    val_ref[...] = mask * changed2 + (1.0 - mask) * g


def _d2(g, hp, damp, W_1, W_2, W_un):
    grid = (B // RB,)
    nb = RB * K2
    return pl.pallas_call(
        _d2_kernel, grid=grid,
        in_specs=[
            pl.BlockSpec((nb, EMB), lambda i: (i, 0)),
            pl.BlockSpec((RB, MSG), lambda i: (i, 0)),
            pl.BlockSpec((RB, K2), lambda i: (i, 0)),
            pl.BlockSpec((EMB + MSG, EMB), lambda i: (0, 0)),
            pl.BlockSpec((EMB, 1), lambda i: (0, 0)),
            pl.BlockSpec((EMB + MSG, EMB), lambda i: (0, 0)),
        ],
        out_specs=pl.BlockSpec((nb, EMB), lambda i: (i, 0)),
        out_shape=jax.ShapeDtypeStruct((B * K2, EMB), jnp.float32),
        name="d2",
    )(g, hp, damp, W_1, W_2, W_un)


def _d4_kernel(s_ref, d_ref, n_ref, pos_ref, neg_ref):
    def norm(x):
        return x / (jnp.sqrt(jnp.sum(x * x, axis=1, keepdims=True)) + EPS)

    sn = norm(s_ref[...])
    dn = norm(d_ref[...])
    nn = norm(n_ref[...])
    pos_ref[...] = jnp.sum(sn * dn, axis=1, keepdims=True)
    neg_ref[...] = jnp.sum(sn * nn, axis=1, keepdims=True)


def _d4(sdn):
    rb = 512
    grid = (B // rb,)
    nblk = B // rb
    return pl.pallas_call(
        _d4_kernel, grid=grid,
        in_specs=[
            pl.BlockSpec((rb, EMB), lambda i: (i, 0)),
            pl.BlockSpec((rb, EMB), lambda i: (nblk + i, 0)),
            pl.BlockSpec((rb, EMB), lambda i: (2 * nblk + i, 0)),
        ],
        out_specs=[
            pl.BlockSpec((rb, 1), lambda i: (i, 0)),
            pl.BlockSpec((rb, 1), lambda i: (i, 0)),
        ],
        out_shape=[
            jax.ShapeDtypeStruct((B, 1), jnp.float32),
            jax.ShapeDtypeStruct((B, 1), jnp.float32),
        ],
        name="d4",
    )(sdn, sdn, sdn)


def kernel(mem, edge_feat, W_g, a, W_e, W_uc, W_un, W_p, W_1, W_2,
           src_idxs, dst_idxs, neg_idxs, edge_idxs, timestamps,
           nbrs_src, nbrs_dst, ts_src, ts_dst,
           up_nbrs_src, up_nbrs_dst, up_ts_src, up_ts_dst):
    i32 = jnp.int32
    src_idxs = src_idxs.astype(i32)
    dst_idxs = dst_idxs.astype(i32)
    neg_idxs = neg_idxs.astype(i32)
    edge_idxs = edge_idxs.astype(i32)

    mem_ref = jax.new_ref(jnp.pad(mem, ((0, NW), (0, 0))))

    idx_g1 = jnp.concatenate([
        nbrs_src.reshape(-1).astype(i32), nbrs_dst.reshape(-1).astype(i32),
        src_idxs, dst_idxs])
    rows1 = _sc_gather(mem_ref, idx_g1, 2 * B * K1 + 2 * B, EMB)
    erows = jnp.take(edge_feat, edge_idxs, axis=0)

    upvals, hp = _d1(rows1, erows,
                     _damp(timestamps, ts_src), _damp(timestamps, ts_dst),
                     W_g, a, W_e, W_uc, W_p)

    _sc_scatter(mem_ref, jnp.concatenate([src_idxs, dst_idxs]),
                upvals.reshape(2 * B, EMB), 2 * B)

    for nbrs, ts in ((up_nbrs_src, up_ts_src), (up_nbrs_dst, up_ts_dst)):
        flat = nbrs.reshape(-1).astype(i32)
        g = _sc_gather(mem_ref, flat, B * K2, EMB)
        vals = _d2(g, hp, _damp(timestamps, ts), W_1, W_2, W_un)
        _sc_scatter(mem_ref, flat, vals, B * K2)

    sdn = _sc_gather(mem_ref, jnp.concatenate([src_idxs, dst_idxs, neg_idxs]),
                     3 * B, EMB)
    pos, neg = _d4(sdn)
    return pos.reshape(B), neg.reshape(B)


# speedup vs baseline: 1.9196x; 1.0364x over previous
import functools

import jax
import jax.numpy as jnp
from jax import lax
from jax.experimental import pallas as pl
from jax.experimental.pallas import tpu as pltpu
from jax.experimental.pallas import tpu_sc as plsc

N_NODES = 100000
EMB = 128
MSG = 128
K1 = 16
K2 = 16
B = 4096
TOT_TIME = 100.0
GAMMA = 1.0
EPS = 1e-10

_INFO = plsc.get_sparse_core_info()
NC = _INFO.num_cores
NS = _INFO.num_subcores
NW = NC * NS
NOWN = (N_NODES + NW - 1) // NW
NPAD = N_NODES + NW
PR_LEN = ((NOWN + 15) // 16) * 16
PR_ITERS = PR_LEN // 16
RC = 512
WL = ((NOWN + RC - 1) // RC) * RC
NCH_MAX = WL // RC
IC = 4096

_mesh = functools.partial(
    plsc.VectorSubcoreMesh, core_axis_name="c", subcore_axis_name="s")
_SC_PARAMS = pltpu.CompilerParams(needs_layout_passes=False)
_PREC = jax.lax.Precision.DEFAULT


def _wid():
    return lax.axis_index("s") * NC + lax.axis_index("c")


def _pick_chunk(per_w, row_bytes):
    target = max(8, (128 * 1024) // row_bytes)
    ch = min(per_w, target)
    while ch > 8 and (per_w % ch or ch % 8):
        ch -= 1
    return ch


def _sc_gather(table, idx, n_rows, dim):
    per_w = n_rows // NW
    ch = _pick_chunk(per_w, dim * 4)
    chunks = per_w // ch

    @functools.partial(
        pl.kernel,
        out_type=jax.ShapeDtypeStruct((n_rows, dim), jnp.float32),
        mesh=_mesh(),
        compiler_params=_SC_PARAMS,
        scratch_types=[
            pltpu.VMEM((ch,), jnp.int32),
            pltpu.VMEM((ch, dim), jnp.float32),
            pltpu.SemaphoreType.DMA,
        ],
        name=f"sc_gather_{n_rows}x{dim}",
    )
    def k(table_hbm, idx_hbm, out_hbm, idxv, rowsv, sem):
        base = _wid() * per_w

        def body(j, carry):
            off = base + j * ch
            pltpu.sync_copy(idx_hbm.at[pl.ds(off, ch)], idxv)
            pltpu.async_copy(table_hbm.at[idxv], rowsv, sem).wait()
            pltpu.sync_copy(rowsv, out_hbm.at[pl.ds(off, ch)])
            return carry

        lax.fori_loop(0, chunks, body, 0)

    return k(table, idx)


def _sc_scatter(mem_ref, idx, val, n_upd):
    ic = min(IC, n_upd)
    chunks = n_upd // ic

    @functools.partial(
        pl.kernel,
        out_type=(),
        mesh=_mesh(),
        compiler_params=_SC_PARAMS,
        scratch_types=[
            pltpu.VMEM((ic,), jnp.int32),
            pltpu.VMEM((ic,), jnp.int32),
            pltpu.VMEM((PR_LEN,), jnp.int32),
            pltpu.VMEM((WL,), jnp.int32),
            pltpu.VMEM((WL,), jnp.int32),
            pltpu.VMEM((RC,), jnp.int32),
            pltpu.VMEM((RC,), jnp.int32),
            pltpu.VMEM((RC, EMB), jnp.float32),
            pltpu.SemaphoreType.DMA,
            pltpu.SemaphoreType.DMA,
            pltpu.SemaphoreType.DMA,
        ],
        name=f"sc_scatter_{n_upd}",
    )
    def k(mem_hbm, idx_hbm, val_hbm, idxv0, idxv1, priov, wposv, wnodev,
          pos_dma, node_dma, rowsv, sem, sem0, sem1):
        wid = _wid()
        base = wid * NOWN
        neg1 = jnp.full((16,), -1, jnp.int32)

        def initb(i, c):
            priov[pl.ds(i * 16, 16)] = neg1
            return c

        lax.fori_loop(0, PR_ITERS, initb, 0)

        bufs = (idxv0, idxv1)
        sems = (sem0, sem1)
        iota16 = lax.iota(jnp.int32, 16)
        unroll = 4
        copies = [
            pltpu.async_copy(idx_hbm.at[pl.ds(c * ic, ic)], bufs[c % 2],
                             sems[c % 2])
            for c in range(min(1, chunks))]
        for c in range(chunks):
            if c + 1 < chunks:
                copies.append(pltpu.async_copy(
                    idx_hbm.at[pl.ds((c + 1) * ic, ic)], bufs[(c + 1) % 2],
                    sems[(c + 1) % 2]))
            copies[c].wait()
            buf = bufs[c % 2]

            def s_body(s, carry2, c=c, buf=buf):
                for u in range(unroll):
                    off = s * (16 * unroll) + u * 16
                    v = buf[pl.ds(off, 16)]
                    pos = c * ic + off + iota16
                    rel = v - base
                    own = rel.astype(jnp.uint32) < jnp.uint32(NOWN)
                    rel_s = jnp.where(own, rel, 0)
                    plsc.store_scatter(priov, [rel_s], pos, mask=own)
                return carry2

            lax.fori_loop(0, ic // (16 * unroll), s_body, 0)

        pad_pos = jnp.full((16,), (wid * 997) % n_upd, jnp.int32)
        dump = jnp.full((16,), N_NODES + wid, jnp.int32)

        def pf(i, c):
            wposv[pl.ds(i * 16, 16)] = pad_pos
            return c

        lax.fori_loop(0, WL // 16, pf, 0)

        def pf2(i, c):
            wnodev[pl.ds(i * 16, 16)] = dump
            return c

        lax.fori_loop(0, WL // 16, pf2, 0)

        def p2(s, cnt):
            p = priov[pl.ds(s * 16, 16)]
            m = p >= 0
            mi = m.astype(jnp.int32)
            pref = plsc.cumsum(mi)
            dest = jnp.where(m, cnt + pref - 1, 0)
            plsc.store_scatter(wposv, [dest], p, mask=m)
            node = s * 16 + lax.iota(jnp.int32, 16) + base
            plsc.store_scatter(wnodev, [dest], node, mask=m)
            return cnt + jnp.sum(mi)

        cnt = lax.fori_loop(0, PR_ITERS, p2, 0)

        nch = (cnt + RC - 1) // RC

        def p3(j, c):
            def stage(i, c2):
                pos_dma[pl.ds(i * 16, 16)] = wposv[pl.ds(j * RC + i * 16, 16)]
                node_dma[pl.ds(i * 16, 16)] = wnodev[pl.ds(j * RC + i * 16, 16)]
                return c2

            lax.fori_loop(0, RC // 16, stage, 0)
            pltpu.async_copy(val_hbm.at[pos_dma], rowsv, sem).wait()
            pltpu.async_copy(rowsv, mem_hbm.at[node_dma], sem).wait()
            return c

        lax.fori_loop(0, nch, p3, 0)

    k(mem_ref, idx, val)


def _damp(timestamps, ts):
    interval = (timestamps[:, None] - ts) / TOT_TIME
    return 1.0 / (1.0 + GAMMA * interval)


RB = 128


def _d1_kernel(nes_ref, ned_ref, ms_ref, md_ref, er_ref, ds_ref, dd_ref,
               wg_ref, a2_ref, we_ref, wuc_ref, wp_ref,
               upv_ref, hp_ref):
    wg = wg_ref[...]

    def phase(ne_ref, m_ref, d_ref):
        ne3 = ne_ref[...].reshape(RB, K1, EMB) * d_ref[...][..., None]
        hn = jnp.dot(ne3.reshape(RB * K1, EMB), wg,
                     preferred_element_type=jnp.float32, precision=_PREC)
        hn3 = hn.reshape(RB, K1, MSG // 2)
        hc = jnp.dot(m_ref[...], wg, preferred_element_type=jnp.float32, precision=_PREC)
        hcb = jnp.broadcast_to(hc[:, None, :], (RB, K1, MSG // 2))
        h_in = jnp.concatenate([hcb, hn3], axis=2).reshape(RB * K1, MSG)
        l3 = jnp.dot(h_in, a2_ref[...],
                     preferred_element_type=jnp.float32, precision=_PREC).reshape(RB, K1, 1)
        l3 = jnp.where(l3 >= 0, l3, 0.2 * l3)
        mx = jnp.max(l3, axis=1, keepdims=True)
        e = jnp.exp(l3 - mx)
        att3 = e / jnp.sum(e, axis=1, keepdims=True)
        return jnp.sum(hn3 * att3, axis=1)

    msg_s = phase(nes_ref, ms_ref, ds_ref)
    msg_d = phase(ned_ref, md_ref, dd_ref)
    hnode = jnp.tanh(jnp.concatenate([msg_s, msg_d], axis=1))
    he = jnp.tanh(jnp.dot(er_ref[...], we_ref[...],
                          preferred_element_type=jnp.float32, precision=_PREC))
    h = jnp.concatenate([hnode, he], axis=1)
    wuc = wuc_ref[...]
    upv_ref[0] = jnp.tanh(jnp.dot(
        jnp.concatenate([ms_ref[...], h], axis=1), wuc,
        preferred_element_type=jnp.float32, precision=_PREC))
    upv_ref[1] = jnp.tanh(jnp.dot(
        jnp.concatenate([md_ref[...], h], axis=1), wuc,
        preferred_element_type=jnp.float32, precision=_PREC))
    hp_ref[...] = jnp.dot(h, wp_ref[...], preferred_element_type=jnp.float32, precision=_PREC)


def _d1(rows1, erows, damp_s, damp_d, W_g, a, W_e, W_uc, W_p):
    grid = (B // RB,)
    nb = RB * K1
    specs = [
        pl.BlockSpec((nb, EMB), lambda i: (i, 0)),
        pl.BlockSpec((nb, EMB), lambda i: (B * K1 // nb + i, 0)),
        pl.BlockSpec((RB, EMB), lambda i: (2 * B * K1 // RB + i, 0)),
        pl.BlockSpec((RB, EMB), lambda i: (2 * B * K1 // RB + B // RB + i, 0)),
        pl.BlockSpec((RB, 16), lambda i: (i, 0)),
        pl.BlockSpec((RB, K1), lambda i: (i, 0)),
        pl.BlockSpec((RB, K1), lambda i: (i, 0)),
        pl.BlockSpec((EMB, MSG // 2), lambda i: (0, 0)),
        pl.BlockSpec((MSG, 1), lambda i: (0, 0)),
        pl.BlockSpec((16, MSG), lambda i: (0, 0)),
        pl.BlockSpec((EMB + 2 * MSG, EMB), lambda i: (0, 0)),
        pl.BlockSpec((2 * MSG, MSG), lambda i: (0, 0)),
    ]
    out_specs = [
        pl.BlockSpec((2, RB, EMB), lambda i: (0, i, 0)),
        pl.BlockSpec((RB, MSG), lambda i: (i, 0)),
    ]
    out_shape = [
        jax.ShapeDtypeStruct((2, B, EMB), jnp.float32),
        jax.ShapeDtypeStruct((B, MSG), jnp.float32),
    ]
    a2 = a.reshape(-1, 1)
    return pl.pallas_call(
        _d1_kernel, grid=grid, in_specs=specs, out_specs=out_specs,
        out_shape=out_shape, name="d1",
    )(rows1, rows1, rows1, rows1, erows, damp_s, damp_d,
      W_g, a2, W_e, W_uc, W_p)


def _d2_kernel(g_ref, hp_ref, d_ref, w1_ref, w2_ref, wun_ref, val_ref):
    g = g_ref[...]
    g3 = g.reshape(RB, K2, EMB)
    ne3 = g3 * d_ref[...][..., None]
    h1 = jnp.broadcast_to(hp_ref[...][:, None, :], (RB, K2, MSG))
    h2 = h1 * ne3
    nrm = jnp.sqrt(jnp.sum(h2 * h2, axis=2, keepdims=True))
    h2n = h2 / (nrm + EPS)
    sc = jnp.sum(h2n, axis=2)
    mx = jnp.max(sc, axis=1, keepdims=True)
    e = jnp.exp(sc - mx)
    att = e / jnp.sum(e, axis=1, keepdims=True)
    changed = h1 * att[..., None]
    ch2d = changed.reshape(RB * K2, MSG)
    x = jnp.concatenate([ne3.reshape(RB * K2, EMB), ch2d], axis=1)
    x = jnp.dot(x, w1_ref[...], preferred_element_type=jnp.float32, precision=_PREC)
    x = jnp.maximum(x, 0.0)
    z = jnp.dot(x, w2_ref[...], preferred_element_type=jnp.float32, precision=_PREC)
    changed2 = jnp.tanh(jnp.dot(
        jnp.concatenate([g, ch2d], axis=1), wun_ref[...],
        preferred_element_type=jnp.float32, precision=_PREC))
    mask = (z >= 0.0).astype(jnp.float32)
    val_ref[...] = mask * changed2 + (1.0 - mask) * g


def _d2(g, hp, damp, W_1, W_2, W_un):
    grid = (B // RB,)
    nb = RB * K2
    return pl.pallas_call(
        _d2_kernel, grid=grid,
        in_specs=[
            pl.BlockSpec((nb, EMB), lambda i: (i, 0)),
            pl.BlockSpec((RB, MSG), lambda i: (i, 0)),
            pl.BlockSpec((RB, K2), lambda i: (i, 0)),
            pl.BlockSpec((EMB + MSG, EMB), lambda i: (0, 0)),
            pl.BlockSpec((EMB, 1), lambda i: (0, 0)),
            pl.BlockSpec((EMB + MSG, EMB), lambda i: (0, 0)),
        ],
        out_specs=pl.BlockSpec((nb, EMB), lambda i: (i, 0)),
        out_shape=jax.ShapeDtypeStruct((B * K2, EMB), jnp.float32),
        name="d2",
    )(g, hp, damp, W_1, W_2, W_un)


def _d4_kernel(s_ref, d_ref, n_ref, pos_ref, neg_ref):
    def norm(x):
        return x / (jnp.sqrt(jnp.sum(x * x, axis=1, keepdims=True)) + EPS)

    sn = norm(s_ref[...])
    dn = norm(d_ref[...])
    nn = norm(n_ref[...])
    pos_ref[...] = jnp.sum(sn * dn, axis=1, keepdims=True)
    neg_ref[...] = jnp.sum(sn * nn, axis=1, keepdims=True)


def _d4(sdn):
    rb = 512
    grid = (B // rb,)
    nblk = B // rb
    return pl.pallas_call(
        _d4_kernel, grid=grid,
        in_specs=[
            pl.BlockSpec((rb, EMB), lambda i: (i, 0)),
            pl.BlockSpec((rb, EMB), lambda i: (nblk + i, 0)),
            pl.BlockSpec((rb, EMB), lambda i: (2 * nblk + i, 0)),
        ],
        out_specs=[
            pl.BlockSpec((rb, 1), lambda i: (i, 0)),
            pl.BlockSpec((rb, 1), lambda i: (i, 0)),
        ],
        out_shape=[
            jax.ShapeDtypeStruct((B, 1), jnp.float32),
            jax.ShapeDtypeStruct((B, 1), jnp.float32),
        ],
        name="d4",
    )(sdn, sdn, sdn)


def kernel(mem, edge_feat, W_g, a, W_e, W_uc, W_un, W_p, W_1, W_2,
           src_idxs, dst_idxs, neg_idxs, edge_idxs, timestamps,
           nbrs_src, nbrs_dst, ts_src, ts_dst,
           up_nbrs_src, up_nbrs_dst, up_ts_src, up_ts_dst):
    i32 = jnp.int32
    src_idxs = src_idxs.astype(i32)
    dst_idxs = dst_idxs.astype(i32)
    neg_idxs = neg_idxs.astype(i32)
    edge_idxs = edge_idxs.astype(i32)

    mem_ref = jax.new_ref(jnp.pad(mem, ((0, NW), (0, 0))))

    idx_g1 = jnp.concatenate([
        nbrs_src.reshape(-1).astype(i32), nbrs_dst.reshape(-1).astype(i32),
        src_idxs, dst_idxs])
    rows1 = _sc_gather(mem_ref, idx_g1, 2 * B * K1 + 2 * B, EMB)
    erows = jnp.take(edge_feat, edge_idxs, axis=0)

    upvals, hp = _d1(rows1, erows,
                     _damp(timestamps, ts_src), _damp(timestamps, ts_dst),
                     W_g, a, W_e, W_uc, W_p)

    _sc_scatter(mem_ref, jnp.concatenate([src_idxs, dst_idxs]),
                upvals.reshape(2 * B, EMB), 2 * B)

    for nbrs, ts in ((up_nbrs_src, up_ts_src), (up_nbrs_dst, up_ts_dst)):
        flat = nbrs.reshape(-1).astype(i32)
        g = _sc_gather(mem_ref, flat, B * K2, EMB)
        vals = _d2(g, hp, _damp(timestamps, ts), W_1, W_2, W_un)
        _sc_scatter(mem_ref, flat, vals, B * K2)

    sdn = _sc_gather(mem_ref, jnp.concatenate([src_idxs, dst_idxs, neg_idxs]),
                     3 * B, EMB)
    pos, neg = _d4(sdn)
    return pos.reshape(B), neg.reshape(B)
